# speedup vs baseline: 6.1662x; 6.1662x over previous
import functools

import jax
import jax.numpy as jnp
from jax import lax
from jax.experimental import pallas as pl
from jax.experimental.pallas import tpu as pltpu
from jax.experimental.pallas import tpu_sc as plsc

VOCAB = 100000
EMB = 128
BATCH = 4096
SEQ = 200

_NC = 2
_NS = 16
_NW = _NC * _NS
_TOTAL = BATCH * SEQ
_PER_W = _TOTAL // _NW
_GRP = 128
_NGRP = _PER_W // _GRP


def _emb_kernel(x_hbm, table_hbm, out_hbm, mask_hbm, idx_v, mask_v, buf_v, sem):
    wid = lax.axis_index("s") * _NC + lax.axis_index("c")
    base = wid * _PER_W

    pltpu.sync_copy(x_hbm.at[wid], idx_v)

    def mask_body(g, carry):
        for j in range(_GRP // 16):
            v = idx_v[g, pl.ds(j * 16, 16)]
            mask_v[g, pl.ds(j * 16, 16)] = jnp.where(
                v != 0, jnp.float32(1.0), jnp.float32(0.0))
        return carry

    lax.fori_loop(0, _NGRP, mask_body, 0)
    pltpu.sync_copy(mask_v, mask_hbm.at[wid])

    def gather_body(g, carry):
        pltpu.async_copy(table_hbm.at[idx_v.at[g]], buf_v, sem).wait()
        pltpu.sync_copy(buf_v, out_hbm.at[pl.ds(base + g * _GRP, _GRP)])
        return carry

    lax.fori_loop(0, _NGRP, gather_body, 0)


@jax.jit
def kernel(x, table):
    x32 = x.astype(jnp.int32).reshape(_NW, _NGRP, _GRP)
    mesh = plsc.VectorSubcoreMesh(core_axis_name="c", subcore_axis_name="s")
    out, mask = pl.kernel(
        _emb_kernel,
        mesh=mesh,
        out_type=[
            jax.ShapeDtypeStruct((_TOTAL, EMB), jnp.float32),
            jax.ShapeDtypeStruct((_NW, _NGRP, _GRP), jnp.float32),
        ],
        scratch_types=[
            pltpu.VMEM((_NGRP, _GRP), jnp.int32),
            pltpu.VMEM((_NGRP, _GRP), jnp.float32),
            pltpu.VMEM((_GRP, EMB), jnp.float32),
            pltpu.SemaphoreType.DMA,
        ],
    )(x32, table)
    return out.reshape(BATCH, SEQ, EMB), mask.reshape(BATCH, SEQ)


# speedup vs baseline: 7.3445x; 1.1911x over previous
import functools

import jax
import jax.numpy as jnp
from jax import lax
from jax.experimental import pallas as pl
from jax.experimental.pallas import tpu as pltpu
from jax.experimental.pallas import tpu_sc as plsc

VOCAB = 100000
EMB = 128
BATCH = 4096
SEQ = 200

_NC = 2
_NS = 16
_NW = _NC * _NS
_TOTAL = BATCH * SEQ
_PER_W = _TOTAL // _NW
_GRP = 128
_NGRP = _PER_W // _GRP


def _emb_kernel(x_hbm, table_hbm, out_hbm, mask_hbm, idx_v, mask_v,
                buf_a, buf_b, gsem, osem):
    wid = lax.axis_index("s") * _NC + lax.axis_index("c")
    base = wid * _PER_W

    pltpu.sync_copy(x_hbm.at[wid], idx_v)

    def out_at(g):
        return out_hbm.at[pl.ds(base + g * _GRP, _GRP)]

    def mask_row(g):
        for j in range(_GRP // 16):
            v = idx_v[g, pl.ds(j * 16, 16)]
            mask_v[g, pl.ds(j * 16, 16)] = jnp.where(
                v != 0, jnp.float32(1.0), jnp.float32(0.0))

    pltpu.async_copy(table_hbm.at[idx_v.at[0]], buf_a, gsem)

    def body(p, carry):
        g0 = 2 * p
        pltpu.make_async_copy(table_hbm.at[idx_v.at[g0]], buf_a, gsem).wait()

        @pl.when(p > 0)
        def _():
            pltpu.make_async_copy(buf_b, out_at(g0 - 1), osem).wait()

        pltpu.async_copy(table_hbm.at[idx_v.at[g0 + 1]], buf_b, gsem)
        pltpu.async_copy(buf_a, out_at(g0), osem)
        mask_row(g0)
        mask_row(g0 + 1)
        pltpu.make_async_copy(table_hbm.at[idx_v.at[g0 + 1]], buf_b, gsem).wait()
        pltpu.make_async_copy(buf_a, out_at(g0), osem).wait()

        @pl.when(g0 + 2 < _NGRP)
        def _():
            pltpu.async_copy(table_hbm.at[idx_v.at[g0 + 2]], buf_a, gsem)

        pltpu.async_copy(buf_b, out_at(g0 + 1), osem)
        return carry

    lax.fori_loop(0, _NGRP // 2, body, 0)
    pltpu.make_async_copy(buf_b, out_at(_NGRP - 1), osem).wait()
    pltpu.sync_copy(mask_v, mask_hbm.at[wid])


@jax.jit
def kernel(x, table):
    x32 = x.astype(jnp.int32).reshape(_NW, _NGRP, _GRP)
    mesh = plsc.VectorSubcoreMesh(core_axis_name="c", subcore_axis_name="s")
    out, mask = pl.kernel(
        _emb_kernel,
        mesh=mesh,
        out_type=[
            jax.ShapeDtypeStruct((_TOTAL, EMB), jnp.float32),
            jax.ShapeDtypeStruct((_NW, _NGRP, _GRP), jnp.float32),
        ],
        scratch_types=[
            pltpu.VMEM((_NGRP, _GRP), jnp.int32),
            pltpu.VMEM((_NGRP, _GRP), jnp.float32),
            pltpu.VMEM((_GRP, EMB), jnp.float32),
            pltpu.VMEM((_GRP, EMB), jnp.float32),
            pltpu.SemaphoreType.DMA,
            pltpu.SemaphoreType.DMA,
        ],
    )(x32, table)
    return out.reshape(BATCH, SEQ, EMB), mask.reshape(BATCH, SEQ)


# speedup vs baseline: 8.8808x; 1.2092x over previous
import functools

import jax
import jax.numpy as jnp
from jax import lax
from jax.experimental import pallas as pl
from jax.experimental.pallas import tpu as pltpu
from jax.experimental.pallas import tpu_sc as plsc

VOCAB = 100000
EMB = 128
BATCH = 4096
SEQ = 200

_NC = 2
_NS = 16
_NW = _NC * _NS
_TOTAL = BATCH * SEQ
_PER_W = _TOTAL // _NW
_GRP = 128
_NGRP = _PER_W // _GRP


_NBUF = 4


def _emb_kernel(x_hbm, table_hbm, out_hbm, mask_hbm, idx_v, mask_v, *rest):
    bufs = rest[:_NBUF]
    gsems = rest[_NBUF:2 * _NBUF]
    osems = rest[2 * _NBUF:3 * _NBUF]
    wid = lax.axis_index("s") * _NC + lax.axis_index("c")
    base = wid * _PER_W

    pltpu.sync_copy(x_hbm.at[wid], idx_v)

    def out_at(g):
        return out_hbm.at[pl.ds(base + g * _GRP, _GRP)]

    def mask_row(g):
        for j in range(_GRP // 16):
            v = idx_v[g, pl.ds(j * 16, 16)]
            mask_v[g, pl.ds(j * 16, 16)] = jnp.where(
                v != 0, jnp.float32(1.0), jnp.float32(0.0))

    for b in range(_NBUF):
        pltpu.async_copy(table_hbm.at[idx_v.at[b]], bufs[b], gsems[b])

    def body(p, carry):
        g0 = p * _NBUF
        for b in range(_NBUF):
            pltpu.make_async_copy(
                table_hbm.at[idx_v.at[g0 + b]], bufs[b], gsems[b]).wait()
            pltpu.async_copy(bufs[b], out_at(g0 + b), osems[b])
        for b in range(_NBUF):
            mask_row(g0 + b)
        for b in range(_NBUF):
            g_next = g0 + _NBUF + b

            @pl.when(g_next < _NGRP)
            def _(b=b, g_next=g_next):
                pltpu.make_async_copy(
                    bufs[b], out_at(g_next - _NBUF), osems[b]).wait()
                pltpu.async_copy(
                    table_hbm.at[idx_v.at[g_next]], bufs[b], gsems[b])
        return carry

    lax.fori_loop(0, _NGRP // _NBUF, body, 0)
    for b in range(_NBUF):
        pltpu.make_async_copy(
            bufs[b], out_at(_NGRP - _NBUF + b), osems[b]).wait()
    pltpu.sync_copy(mask_v, mask_hbm.at[wid])


@jax.jit
def kernel(x, table):
    x32 = x.astype(jnp.int32).reshape(_NW, _NGRP, _GRP)
    mesh = plsc.VectorSubcoreMesh(core_axis_name="c", subcore_axis_name="s")
    out, mask = pl.kernel(
        _emb_kernel,
        mesh=mesh,
        out_type=[
            jax.ShapeDtypeStruct((_TOTAL, EMB), jnp.float32),
            jax.ShapeDtypeStruct((_NW, _NGRP, _GRP), jnp.float32),
        ],
        scratch_types=[
            pltpu.VMEM((_NGRP, _GRP), jnp.int32),
            pltpu.VMEM((_NGRP, _GRP), jnp.float32),
            *[pltpu.VMEM((_GRP, EMB), jnp.float32) for _ in range(_NBUF)],
            *[pltpu.SemaphoreType.DMA for _ in range(2 * _NBUF)],
        ],
    )(x32, table)
    return out.reshape(BATCH, SEQ, EMB), mask.reshape(BATCH, SEQ)


# speedup vs baseline: 9.0086x; 1.0144x over previous
import functools

import jax
import jax.numpy as jnp
from jax import lax
from jax.experimental import pallas as pl
from jax.experimental.pallas import tpu as pltpu
from jax.experimental.pallas import tpu_sc as plsc

VOCAB = 100000
EMB = 128
BATCH = 4096
SEQ = 200

_NC = 2
_NS = 16
_NW = _NC * _NS
_TOTAL = BATCH * SEQ
_PER_W = _TOTAL // _NW
_GRP = 128
_NGRP = _PER_W // _GRP
_NBUF = 5


def _emb_kernel(x_hbm, table_hbm, out_hbm, idx_v, *rest):
    bufs = rest[:_NBUF]
    gsems = rest[_NBUF:2 * _NBUF]
    osems = rest[2 * _NBUF:3 * _NBUF]
    wid = lax.axis_index("s") * _NC + lax.axis_index("c")
    base = wid * _PER_W

    pltpu.sync_copy(x_hbm.at[wid], idx_v)

    def out_at(g):
        return out_hbm.at[pl.ds(base + g * _GRP, _GRP)]

    for b in range(_NBUF):
        pltpu.async_copy(table_hbm.at[idx_v.at[b]], bufs[b], gsems[b])

    def body(p, carry):
        g0 = p * _NBUF
        for b in range(_NBUF):
            pltpu.make_async_copy(
                table_hbm.at[idx_v.at[g0 + b]], bufs[b], gsems[b]).wait()
            pltpu.async_copy(bufs[b], out_at(g0 + b), osems[b])
        for b in range(_NBUF):
            g_next = g0 + _NBUF + b

            @pl.when(g_next < _NGRP)
            def _(b=b, g_next=g_next):
                pltpu.make_async_copy(
                    bufs[b], out_at(g_next - _NBUF), osems[b]).wait()
                pltpu.async_copy(
                    table_hbm.at[idx_v.at[g_next]], bufs[b], gsems[b])
        return carry

    lax.fori_loop(0, _NGRP // _NBUF, body, 0)
    for b in range(_NBUF):
        pltpu.make_async_copy(
            bufs[b], out_at(_NGRP - _NBUF + b), osems[b]).wait()


def _mask_kernel(x_ref, o_ref):
    o_ref[...] = jnp.where(x_ref[...] != 0,
                           jnp.float32(1.0), jnp.float32(0.0))


@jax.jit
def kernel(x, table):
    x32 = x.astype(jnp.int32)
    xw = x32.reshape(_NW, _NGRP, _GRP)
    mesh = plsc.VectorSubcoreMesh(core_axis_name="c", subcore_axis_name="s")
    out = pl.kernel(
        _emb_kernel,
        mesh=mesh,
        out_type=jax.ShapeDtypeStruct((_TOTAL, EMB), jnp.float32),
        scratch_types=[
            pltpu.VMEM((_NGRP, _GRP), jnp.int32),
            *[pltpu.VMEM((_GRP, EMB), jnp.float32) for _ in range(_NBUF)],
            *[pltpu.SemaphoreType.DMA for _ in range(2 * _NBUF)],
        ],
    )(xw, table)
    mask = pl.pallas_call(
        _mask_kernel,
        out_shape=jax.ShapeDtypeStruct((BATCH, SEQ), jnp.float32),
        grid=(8,),
        in_specs=[pl.BlockSpec((BATCH // 8, SEQ), lambda i: (i, 0))],
        out_specs=pl.BlockSpec((BATCH // 8, SEQ), lambda i: (i, 0)),
    )(x32)
    return out.reshape(BATCH, SEQ, EMB), mask


# speedup vs baseline: 9.0610x; 1.0058x over previous
import functools

import jax
import jax.numpy as jnp
from jax import lax
from jax.experimental import pallas as pl
from jax.experimental.pallas import tpu as pltpu
from jax.experimental.pallas import tpu_sc as plsc

VOCAB = 100000
EMB = 128
BATCH = 4096
SEQ = 200

_NC = 2
_NS = 16
_NW = _NC * _NS
_TOTAL = BATCH * SEQ
_PER_W = _TOTAL // _NW
_GRP = 128
_NGRP = _PER_W // _GRP
_NBUF = 3
_SLAB = 2 * _GRP
_NSTEP = _PER_W // _SLAB


def _emb_kernel(x_hbm, table_hbm, out_hbm, idx_v, *rest):
    bufs = rest[:_NBUF]
    gsems = rest[_NBUF:2 * _NBUF]
    osems = rest[2 * _NBUF:3 * _NBUF]
    wid = lax.axis_index("s") * _NC + lax.axis_index("c")
    base = wid * _PER_W

    pltpu.sync_copy(x_hbm.at[wid], idx_v)

    def gathers(s, b, start):
        for h in range(2):
            cp = pltpu.make_async_copy(
                table_hbm.at[idx_v.at[2 * s + h]],
                bufs[b].at[pl.ds(h * _GRP, _GRP)], gsems[b])
            cp.start() if start else cp.wait()

    def out(s, b, start):
        cp = pltpu.make_async_copy(
            bufs[b], out_hbm.at[pl.ds(base + s * _SLAB, _SLAB)], osems[b])
        cp.start() if start else cp.wait()

    for b in range(_NBUF):
        gathers(b, b, True)

    def body(p, carry):
        s0 = p * _NBUF
        for b in range(_NBUF):
            gathers(s0 + b, b, False)
            out(s0 + b, b, True)
        for b in range(_NBUF):
            s_next = s0 + _NBUF + b

            @pl.when(s_next < _NSTEP)
            def _(b=b, s_next=s_next):
                out(s_next - _NBUF, b, False)
                gathers(s_next, b, True)
        return carry

    lax.fori_loop(0, _NSTEP // _NBUF, body, 0)
    gathers(_NSTEP - 1, 0, False)
    out(_NSTEP - 1, 0, True)
    out(_NSTEP - 3, 1, False)
    out(_NSTEP - 2, 2, False)
    out(_NSTEP - 1, 0, False)


def _mask_kernel(x_ref, o_ref):
    o_ref[...] = jnp.where(x_ref[...] != 0,
                           jnp.float32(1.0), jnp.float32(0.0))


@jax.jit
def kernel(x, table):
    x32 = x.astype(jnp.int32)
    xw = x32.reshape(_NW, _NGRP, _GRP)
    mesh = plsc.VectorSubcoreMesh(core_axis_name="c", subcore_axis_name="s")
    out = pl.kernel(
        _emb_kernel,
        mesh=mesh,
        out_type=jax.ShapeDtypeStruct((_TOTAL, EMB), jnp.float32),
        scratch_types=[
            pltpu.VMEM((_NGRP, _GRP), jnp.int32),
            *[pltpu.VMEM((_SLAB, EMB), jnp.float32) for _ in range(_NBUF)],
            *[pltpu.SemaphoreType.DMA for _ in range(2 * _NBUF)],
        ],
    )(xw, table)
    mask = pl.pallas_call(
        _mask_kernel,
        out_shape=jax.ShapeDtypeStruct((BATCH, SEQ), jnp.float32),
        grid=(8,),
        in_specs=[pl.BlockSpec((BATCH // 8, SEQ), lambda i: (i, 0))],
        out_specs=pl.BlockSpec((BATCH // 8, SEQ), lambda i: (i, 0)),
    )(x32)
    return out.reshape(BATCH, SEQ, EMB), mask


# speedup vs baseline: 9.0651x; 1.0005x over previous
import jax
import jax.numpy as jnp
from jax import lax
from jax.experimental import pallas as pl
from jax.experimental.pallas import tpu as pltpu
from jax.experimental.pallas import tpu_sc as plsc

VOCAB = 100000
EMB = 128
BATCH = 4096
SEQ = 200

_NC = 2
_NS = 16
_NW = _NC * _NS
_TOTAL = BATCH * SEQ
_PER_W = _TOTAL // _NW
_GRP = 128
_NGRP = _PER_W // _GRP
_NBUF = 3
_SLAB = 2 * _GRP
_NSTEP = _PER_W // _SLAB


def _emb_kernel(x_hbm, table_hbm, out_hbm, idx_v, *rest):
    bufs = rest[:_NBUF]
    gsems = rest[_NBUF:2 * _NBUF]
    osems = rest[2 * _NBUF:3 * _NBUF]
    wid = lax.axis_index("s") * _NC + lax.axis_index("c")
    base = wid * _PER_W

    pltpu.sync_copy(x_hbm.at[wid], idx_v)

    def gathers(s, b, start):
        for h in range(2):
            cp = pltpu.make_async_copy(
                table_hbm.at[idx_v.at[2 * s + h]],
                bufs[b].at[pl.ds(h * _GRP, _GRP)], gsems[b])
            cp.start() if start else cp.wait()

    def out(s, b, start):
        cp = pltpu.make_async_copy(
            bufs[b], out_hbm.at[pl.ds(base + s * _SLAB, _SLAB)], osems[b])
        cp.start() if start else cp.wait()

    for b in range(_NBUF):
        gathers(b, b, True)

    def body(p, carry):
        s0 = p * _NBUF
        for b in range(_NBUF):
            gathers(s0 + b, b, False)
            out(s0 + b, b, True)
        for b in range(_NBUF):
            s_next = s0 + _NBUF + b

            @pl.when(s_next < _NSTEP)
            def _(b=b, s_next=s_next):
                out(s_next - _NBUF, b, False)
                gathers(s_next, b, True)
        return carry

    lax.fori_loop(0, _NSTEP // _NBUF, body, 0)
    gathers(_NSTEP - 1, 0, False)
    out(_NSTEP - 1, 0, True)
    out(_NSTEP - 3, 1, False)
    out(_NSTEP - 2, 2, False)
    out(_NSTEP - 1, 0, False)


def _mask_kernel(x_ref, o_ref):
    o_ref[...] = jnp.where(x_ref[...] != 0,
                           jnp.float32(1.0), jnp.float32(0.0))


@jax.jit
def kernel(x, table):
    x32 = x.astype(jnp.int32)
    xw = x32.reshape(_NW, _NGRP, _GRP)
    mesh = plsc.VectorSubcoreMesh(core_axis_name="c", subcore_axis_name="s")
    out = pl.kernel(
        _emb_kernel,
        mesh=mesh,
        out_type=jax.ShapeDtypeStruct((_TOTAL, EMB), jnp.float32),
        scratch_types=[
            pltpu.VMEM((_NGRP, _GRP), jnp.int32),
            *[pltpu.VMEM((_SLAB, EMB), jnp.float32) for _ in range(_NBUF)],
            *[pltpu.SemaphoreType.DMA for _ in range(2 * _NBUF)],
        ],
    )(xw, table)
    mask = pl.pallas_call(
        _mask_kernel,
        out_shape=jax.ShapeDtypeStruct((BATCH, SEQ), jnp.float32),
        grid=(8,),
        in_specs=[pl.BlockSpec((BATCH // 8, SEQ), lambda i: (i, 0))],
        out_specs=pl.BlockSpec((BATCH // 8, SEQ), lambda i: (i, 0)),
    )(x32)
    return out.reshape(BATCH, SEQ, EMB), mask
